# baseline (speedup 1.0000x reference)
import functools

import jax
import jax.numpy as jnp
from jax.experimental import pallas as pl
from jax.experimental.pallas import tpu as pltpu
from jax.sharding import Mesh, PartitionSpec as P

_BATCH = 32
_VOCAB = 1_000_000
_CHUNK = 8192

_K0 = 0
_K1 = 42
_KS2 = 0x1BD11BDA ^ _K0 ^ _K1
_TINY = float(jnp.finfo(jnp.float32).tiny)

_ROT = ((13, 15, 26, 6), (17, 29, 16, 24))


def _rotl(x, r):
    return (x << jnp.uint32(r)) | (x >> jnp.uint32(32 - r))


def _threefry_bits(flat):
    ks = (jnp.uint32(_K0), jnp.uint32(_K1), jnp.uint32(_KS2))
    x0 = jnp.full_like(flat, ks[0])
    x1 = flat + ks[1]
    for i in range(5):
        for r in _ROT[i % 2]:
            x0 = x0 + x1
            x1 = _rotl(x1, r) ^ x0
        x0 = x0 + ks[(i + 1) % 3]
        x1 = x1 + ks[(i + 2) % 3] + jnp.uint32(i + 1)
    return x0 ^ x1


def _gumbel_from_bits(bits):
    fb = (bits >> jnp.uint32(9)) | jnp.uint32(0x3F800000)
    f = jax.lax.bitcast_convert_type(fb, jnp.float32) - jnp.float32(1.0)
    tiny = jnp.float32(_TINY)
    u = jnp.maximum(tiny, f * (jnp.float32(1.0) - tiny) + tiny)
    return -jnp.log(-jnp.log(u))


def _sample_kernel(roff_ref, x_ref, o_ref, acc_val, acc_step, *,
                   nsteps, local_v, nrows):
    j = pl.program_id(0)

    @pl.when(j == 0)
    def _init():
        acc_val[...] = jnp.full((nrows, _CHUNK), -jnp.inf, jnp.float32)
        acc_step[...] = jnp.zeros((nrows, _CHUNK), jnp.int32)

    row = jax.lax.broadcasted_iota(jnp.uint32, (nrows, _CHUNK), 0)
    col = jax.lax.broadcasted_iota(jnp.uint32, (nrows, _CHUNK), 1)
    roff = roff_ref[0].astype(jnp.uint32)
    flat = (row + roff) * jnp.uint32(_VOCAB) + col \
        + j.astype(jnp.uint32) * jnp.uint32(_CHUNK)

    g = _gumbel_from_bits(_threefry_bits(flat))
    val = x_ref[...] + g
    lim = local_v - j * _CHUNK
    icol = jax.lax.broadcasted_iota(jnp.int32, (nrows, _CHUNK), 1)
    val = jnp.where(icol < lim, val, -jnp.inf)

    take = val > acc_val[...]
    acc_val[...] = jnp.where(take, val, acc_val[...])
    acc_step[...] = jnp.where(take, j, acc_step[...])

    @pl.when(j == nsteps - 1)
    def _finish():
        av = acc_val[...]
        m = jnp.max(av, axis=1, keepdims=True)
        idx = acc_step[...] * _CHUNK + jax.lax.broadcasted_iota(
            jnp.int32, (nrows, _CHUNK), 1)
        cand = jnp.where(av == m, idx, jnp.int32(0x7FFFFFFF))
        o_ref[...] = jnp.min(cand, axis=1, keepdims=True)


def _sample(x, row_off):
    nrows = x.shape[0]
    nsteps = (_VOCAB + _CHUNK - 1) // _CHUNK
    roff_arr = jnp.full((1,), row_off, jnp.int32)
    return pl.pallas_call(
        functools.partial(_sample_kernel, nsteps=nsteps, local_v=_VOCAB,
                          nrows=nrows),
        grid=(nsteps,),
        in_specs=[
            pl.BlockSpec(memory_space=pltpu.SMEM),
            pl.BlockSpec((nrows, _CHUNK), lambda j: (0, j)),
        ],
        out_specs=pl.BlockSpec((nrows, 1), lambda j: (0, 0)),
        out_shape=jax.ShapeDtypeStruct((nrows, 1), jnp.int32),
        scratch_shapes=[
            pltpu.VMEM((nrows, _CHUNK), jnp.float32),
            pltpu.VMEM((nrows, _CHUNK), jnp.int32),
        ],
    )(roff_arr, x)


def kernel(logits):
    devs = jax.devices()
    if len(devs) >= 2 and logits.shape[0] % 2 == 0:
        mesh = Mesh(devs[:2], ("x",))
        rows_per = logits.shape[0] // 2

        def shard_fn(x):
            roff = jax.lax.axis_index("x").astype(jnp.int32) * rows_per
            return _sample(x, roff)

        return jax.shard_map(
            shard_fn,
            mesh=mesh,
            in_specs=P("x", None),
            out_specs=P("x", None),
            check_vma=False,
        )(logits)
    return _sample(logits, jnp.int32(0))
